# baseline (speedup 1.0000x reference)
import jax
import jax.numpy as jnp
from jax.experimental import pallas as pl
from jax.experimental.pallas import tpu as pltpu

_TB = 8


def _fused_body(TB, Bs, I, H, O, n_steps):
    f32 = jnp.float32
    bf16 = jnp.bfloat16
    R = TB * Bs

    def body(xs_ref, h0_ref, wcat_ref, bcat_ref, whh_ref, mw_ref, wouh_ref,
             out_ref, hlast_ref, stA, stB, zlA, zlB, hc):
        c = pl.program_id(0)

        @pl.when(c == 0)
        def _():
            hc[...] = h0_ref[...]

        def head(st, zl, sl):
            logits = (zl[...]
                      + jnp.dot(st[0:R, :], mw_ref[...],
                                preferred_element_type=f32)
                      + jnp.dot(st[Bs:R + Bs, :], wouh_ref[...],
                                preferred_element_type=f32))
            mx = jnp.max(logits, axis=-1, keepdims=True)
            y = logits - mx
            lse = jnp.log(jnp.sum(jnp.exp(y), axis=-1, keepdims=True))
            out_ref[sl] = (y - lse).reshape(TB, Bs, O)

        head(stA, zlA, slice(0, TB))
        head(stB, zlB, slice(TB, 2 * TB))

        x = xs_ref[...].reshape(2 * R, I).astype(bf16)
        zA = jnp.dot(x[0:R], wcat_ref[...],
                     preferred_element_type=f32) + bcat_ref[...]
        zB = jnp.dot(x[R:2 * R], wcat_ref[...],
                     preferred_element_type=f32) + bcat_ref[...]

        NS = 2
        Bq = Bs // NS
        h = hc[...]
        parts = [h[j * Bq:(j + 1) * Bq] for j in range(NS)]
        whh = whh_ref[...]
        for st, z in ((stA, zA), (stB, zB)):
            for j in range(NS):
                st[j * Bq:(j + 1) * Bq, :] = parts[j].astype(bf16)
            for i in range(TB):
                for j in range(NS):
                    hbj = parts[j].astype(bf16)
                    parts[j] = z[i * Bs + j * Bq:i * Bs + (j + 1) * Bq, :H] + \
                        jnp.dot(hbj, whh, preferred_element_type=f32)
                    st[(i + 1) * Bs + j * Bq:(i + 1) * Bs + (j + 1) * Bq,
                       :] = parts[j].astype(bf16)

        zlA[...] = zA[:, H:]
        zlB[...] = zB[:, H:]

        @pl.when(c < n_steps)
        def _():
            for j in range(NS):
                hc[j * Bq:(j + 1) * Bq, :] = parts[j]
                hlast_ref[j * Bq:(j + 1) * Bq, :] = parts[j]

    return body


def _rnn_scan(xs, h0, wcat, bcat, whh, m_w, wouh, H, O):
    T, Bs, I = xs.shape
    f32 = jnp.float32
    bf16 = jnp.bfloat16
    TB = _TB
    n_steps = T // (2 * TB)
    IO = wcat.shape[1]
    last = n_steps - 1

    return pl.pallas_call(
        _fused_body(TB, Bs, I, H, O, n_steps),
        grid=(n_steps + 1,),
        in_specs=[
            pl.BlockSpec((2 * TB, Bs, I),
                         lambda c: (jnp.minimum(c, last), 0, 0)),
            pl.BlockSpec((Bs, H), lambda c: (0, 0)),
            pl.BlockSpec((I, IO), lambda c: (0, 0)),
            pl.BlockSpec((1, IO), lambda c: (0, 0)),
            pl.BlockSpec((H, H), lambda c: (0, 0)),
            pl.BlockSpec((H, O), lambda c: (0, 0)),
            pl.BlockSpec((H, O), lambda c: (0, 0)),
        ],
        out_specs=[
            pl.BlockSpec((2 * TB, Bs, O),
                         lambda c: (jnp.maximum(c - 1, 0), 0, 0)),
            pl.BlockSpec((Bs, H), lambda c: (0, 0)),
        ],
        out_shape=(
            jax.ShapeDtypeStruct((T, Bs, O), f32),
            jax.ShapeDtypeStruct((Bs, H), f32),
        ),
        scratch_shapes=[
            pltpu.VMEM(((TB + 1) * Bs, H), bf16),
            pltpu.VMEM(((TB + 1) * Bs, H), bf16),
            pltpu.VMEM((TB * Bs, O), f32),
            pltpu.VMEM((TB * Bs, O), f32),
            pltpu.VMEM((Bs, H), f32),
        ],
        compiler_params=pltpu.CompilerParams(
            dimension_semantics=("arbitrary",),
        ),
    )(xs, h0, wcat, bcat, whh, m_w, wouh)


def kernel(xs, h0, wih_x, b_ih, wio_x, b_io, whh, wio_h, wou_o, wou_h, bou):
    T, B, I = xs.shape
    H = whh.shape[0]
    O = wou_o.shape[0]
    f32 = jnp.float32
    bf16 = jnp.bfloat16

    wou_f = wou_o.astype(f32)
    wfold = jnp.dot(wio_x, wou_f)
    bfold = jnp.dot(b_io, wou_f) + bou[0]
    wcat = jnp.concatenate([wih_x, wfold], axis=1).astype(bf16)
    bcat = jnp.concatenate([b_ih, bfold]).reshape(1, H + O)
    m_w = jnp.dot(wio_h.astype(f32), wou_f).astype(bf16)

    return _rnn_scan(xs, h0, wcat, bcat, whh, m_w, wouh=wou_h, H=H, O=O)


# speedup vs baseline: 1.0447x; 1.0447x over previous
import jax
import jax.numpy as jnp
from jax.experimental import pallas as pl
from jax.experimental.pallas import tpu as pltpu

_TB = 8
_NS = 2
_RB = 256


def _fused_body(TB, Bs, I, H, O, n_steps):
    f32 = jnp.float32
    bf16 = jnp.bfloat16
    R = TB * Bs

    def body(xs_ref, h0_ref, wcat_ref, bcat_ref, whh_ref, mw_ref, wouh_ref,
             out_ref, hlast_ref, stA, stB, zoA, zoB, zh, hc):
        c = pl.program_id(0)

        @pl.when(c == 0)
        def _():
            hc[...] = h0_ref[...]

        tpb = _RB // Bs
        for st, zo, base in ((stA, zoA, 0), (stB, zoB, TB)):
            for r in range(R // _RB):
                r0 = r * _RB
                r1 = r0 + _RB
                logits = (zo[r0:r1, :]
                          + jnp.dot(st[r0:r1, :], mw_ref[...],
                                    preferred_element_type=f32)
                          + jnp.dot(st[Bs + r0:Bs + r1, :], wouh_ref[...],
                                    preferred_element_type=f32))
                mx = jnp.max(logits, axis=-1, keepdims=True)
                y = logits - mx
                lse = jnp.log(jnp.sum(jnp.exp(y), axis=-1, keepdims=True))
                t0 = base + r * tpb
                out_ref[t0:t0 + tpb] = (y - lse).reshape(tpb, Bs, O)

        x = xs_ref[...].reshape(2 * R, I).astype(bf16)
        zh[...] = jnp.dot(x, wcat_ref[:, 0:H],
                          preferred_element_type=f32) + bcat_ref[:, 0:H]
        zoA[...] = jnp.dot(x[0:R], wcat_ref[:, H:],
                           preferred_element_type=f32) + bcat_ref[:, H:]
        zoB[...] = jnp.dot(x[R:2 * R], wcat_ref[:, H:],
                           preferred_element_type=f32) + bcat_ref[:, H:]

        Bq = Bs // _NS
        h = hc[...]
        parts = [h[j * Bq:(j + 1) * Bq] for j in range(_NS)]
        whh = whh_ref[...]
        for st, zbase in ((stA, 0), (stB, R)):
            for j in range(_NS):
                st[j * Bq:(j + 1) * Bq, :] = parts[j].astype(bf16)
            for i in range(TB):
                for j in range(_NS):
                    hbj = parts[j].astype(bf16)
                    rows = zbase + i * Bs + j * Bq
                    parts[j] = zh[rows:rows + Bq, :] + jnp.dot(
                        hbj, whh, preferred_element_type=f32)
                    st[(i + 1) * Bs + j * Bq:(i + 1) * Bs + (j + 1) * Bq,
                       :] = parts[j].astype(bf16)

        @pl.when(c < n_steps)
        def _():
            for j in range(_NS):
                hc[j * Bq:(j + 1) * Bq, :] = parts[j]
                hlast_ref[j * Bq:(j + 1) * Bq, :] = parts[j]

    return body


def _rnn_scan(xs, h0, wcat, bcat, whh, m_w, wouh, H, O):
    T, Bs, I = xs.shape
    f32 = jnp.float32
    bf16 = jnp.bfloat16
    TB = _TB
    n_steps = T // (2 * TB)
    IO = wcat.shape[1]
    R = TB * Bs
    last = n_steps - 1

    return pl.pallas_call(
        _fused_body(TB, Bs, I, H, O, n_steps),
        grid=(n_steps + 1,),
        in_specs=[
            pl.BlockSpec((2 * TB, Bs, I),
                         lambda c: (jnp.minimum(c, last), 0, 0)),
            pl.BlockSpec((Bs, H), lambda c: (0, 0)),
            pl.BlockSpec((I, IO), lambda c: (0, 0)),
            pl.BlockSpec((1, IO), lambda c: (0, 0)),
            pl.BlockSpec((H, H), lambda c: (0, 0)),
            pl.BlockSpec((H, O), lambda c: (0, 0)),
            pl.BlockSpec((H, O), lambda c: (0, 0)),
        ],
        out_specs=[
            pl.BlockSpec((2 * TB, Bs, O),
                         lambda c: (jnp.maximum(c - 1, 0), 0, 0)),
            pl.BlockSpec((Bs, H), lambda c: (0, 0)),
        ],
        out_shape=(
            jax.ShapeDtypeStruct((T, Bs, O), f32),
            jax.ShapeDtypeStruct((Bs, H), f32),
        ),
        scratch_shapes=[
            pltpu.VMEM(((TB + 1) * Bs, H), bf16),
            pltpu.VMEM(((TB + 1) * Bs, H), bf16),
            pltpu.VMEM((R, O), f32),
            pltpu.VMEM((R, O), f32),
            pltpu.VMEM((2 * R, H), f32),
            pltpu.VMEM((Bs, H), f32),
        ],
        compiler_params=pltpu.CompilerParams(
            dimension_semantics=("arbitrary",),
        ),
    )(xs, h0, wcat, bcat, whh, m_w, wouh)


def kernel(xs, h0, wih_x, b_ih, wio_x, b_io, whh, wio_h, wou_o, wou_h, bou):
    T, B, I = xs.shape
    H = whh.shape[0]
    O = wou_o.shape[0]
    f32 = jnp.float32
    bf16 = jnp.bfloat16

    wou_f = wou_o.astype(f32)
    wfold = jnp.dot(wio_x, wou_f)
    bfold = jnp.dot(b_io, wou_f) + bou[0]
    wcat = jnp.concatenate([wih_x, wfold], axis=1).astype(bf16)
    bcat = jnp.concatenate([b_ih, bfold]).reshape(1, H + O)
    m_w = jnp.dot(wio_h.astype(f32), wou_f).astype(bf16)

    return _rnn_scan(xs, h0, wcat, bcat, whh, m_w, wouh=wou_h, H=H, O=O)
